# Initial kernel scaffold; baseline (speedup 1.0000x reference)
import jax
import jax.numpy as jnp
from jax.experimental import pallas as pl


def kernel(x, edge_index, W1, b1, W2, b2, Wfc, bfc):
    raise NotImplementedError("write your pallas kernel here")


# speedup vs baseline: 9.0927x; 9.0927x over previous
import functools

import jax
import jax.numpy as jnp
from jax import lax
from jax.experimental import pallas as pl
from jax.experimental.pallas import tpu as pltpu
from jax.experimental.pallas import tpu_sc as plsc

N = 10000
F = 128
H = 256
CLS = 40
E = 320000
CH = 128
NT = 16
NC = 2
NPAD = 10112
RPT = NPAD // NT
ECH = 2560
EPAD = ECH * CH
NB = 16

_MESH = plsc.VectorSubcoreMesh(
    core_axis_name="c", subcore_axis_name="s", num_cores=NC, num_subcores=NT)


@functools.partial(
    pl.kernel,
    out_type=jax.ShapeDtypeStruct((NC * NPAD, 16), jnp.float32),
    mesh=_MESH,
    scratch_types=[
        pltpu.VMEM((NB, CH), jnp.int32),
        pltpu.VMEM((CH, 16), jnp.float32),
        pltpu.VMEM_SHARED((NPAD, 16), jnp.float32),
    ],
    compiler_params=pltpu.CompilerParams(use_tc_tiling_on_sc=False),
)
def _sc_degree(dst_hbm, ones_hbm, zeros_hbm, out, didx, ones_v, acc):
    c = lax.axis_index("c")
    s = lax.axis_index("s")
    wid = c * NT + s
    r0 = s * RPT
    nch = ECH // (NC * NT)
    pltpu.sync_copy(zeros_hbm.at[pl.ds(r0, RPT)], acc.at[pl.ds(r0, RPT)])
    pltpu.sync_copy(ones_hbm, ones_v)
    plsc.subcore_barrier()

    def outer(b, carry):
        pltpu.sync_copy(dst_hbm.at[pl.ds(wid * nch + b * NB, NB)], didx)

        def body(j, carry2):
            pltpu.sync_copy(ones_v, acc.at[didx.at[j]], add=True)
            return carry2

        lax.fori_loop(0, NB, body, None)
        return carry

    lax.fori_loop(0, nch // NB, outer, None)
    plsc.subcore_barrier()
    pltpu.sync_copy(acc.at[pl.ds(r0, RPT)], out.at[pl.ds(c * NPAD + r0, RPT)])


@functools.partial(
    pl.kernel,
    out_type=jax.ShapeDtypeStruct((NC * NPAD, F), jnp.float32),
    mesh=_MESH,
    scratch_types=[
        pltpu.VMEM((NB, CH), jnp.int32),
        pltpu.VMEM((NB, CH), jnp.int32),
        pltpu.VMEM((CH, F), jnp.float32),
        pltpu.VMEM_SHARED((NPAD, F), jnp.float32),
    ],
)
def _sc_edge_sum_split(tab_hbm, src_hbm, dst_hbm, zeros_hbm, out,
                       sidx, didx, rows, acc):
    c = lax.axis_index("c")
    s = lax.axis_index("s")
    wid = c * NT + s
    r0 = s * RPT
    nch = ECH // (NC * NT)
    pltpu.sync_copy(zeros_hbm.at[pl.ds(r0, RPT)], acc.at[pl.ds(r0, RPT)])
    plsc.subcore_barrier()

    def outer(b, carry):
        pltpu.sync_copy(src_hbm.at[pl.ds(wid * nch + b * NB, NB)], sidx)
        pltpu.sync_copy(dst_hbm.at[pl.ds(wid * nch + b * NB, NB)], didx)

        def body(j, carry2):
            pltpu.sync_copy(tab_hbm.at[sidx.at[j]], rows)
            pltpu.sync_copy(rows, acc.at[didx.at[j]], add=True)
            return carry2

        lax.fori_loop(0, NB, body, None)
        return carry

    lax.fori_loop(0, nch // NB, outer, None)
    plsc.subcore_barrier()
    pltpu.sync_copy(acc.at[pl.ds(r0, RPT)], out.at[pl.ds(c * NPAD + r0, RPT)])


@functools.partial(
    pl.kernel,
    out_type=jax.ShapeDtypeStruct((NC * NPAD, F), jnp.float32),
    mesh=_MESH,
    scratch_types=[
        pltpu.VMEM((NB, CH), jnp.int32),
        pltpu.VMEM((NB, CH), jnp.int32),
        pltpu.VMEM((CH, F), jnp.float32),
        pltpu.VMEM_SHARED((NPAD, F), jnp.float32),
    ],
)
def _sc_edge_sum_sharded(tabs_hbm, srcs_hbm, dst_hbm, zeros_hbm, out,
                         sidx, didx, rows, acc):
    c = lax.axis_index("c")
    s = lax.axis_index("s")
    r0 = s * RPT
    nch = ECH // NT
    pltpu.sync_copy(zeros_hbm.at[pl.ds(r0, RPT)], acc.at[pl.ds(r0, RPT)])
    plsc.subcore_barrier()

    def outer(b, carry):
        pltpu.sync_copy(srcs_hbm.at[pl.ds(c * ECH + s * nch + b * NB, NB)], sidx)
        pltpu.sync_copy(dst_hbm.at[pl.ds(s * nch + b * NB, NB)], didx)

        def body(j, carry2):
            pltpu.sync_copy(tabs_hbm.at[sidx.at[j]], rows)
            pltpu.sync_copy(rows, acc.at[didx.at[j]], add=True)
            return carry2

        lax.fori_loop(0, NB, body, None)
        return carry

    lax.fori_loop(0, nch // NB, outer, None)
    plsc.subcore_barrier()
    pltpu.sync_copy(acc.at[pl.ds(r0, RPT)], out.at[pl.ds(c * NPAD + r0, RPT)])


BN = 2000


def _tc_prescale_body(d0, d1, x, o):
    dinv = lax.rsqrt(d0[...] + d1[...] + 1.0)
    o[...] = x[...] * dinv


_tc_prescale = pl.pallas_call(
    _tc_prescale_body,
    grid=(N // BN,),
    in_specs=[
        pl.BlockSpec((BN, 1), lambda i: (i, 0)),
        pl.BlockSpec((BN, 1), lambda i: (i, 0)),
        pl.BlockSpec((BN, F), lambda i: (i, 0)),
    ],
    out_specs=pl.BlockSpec((BN, F), lambda i: (i, 0)),
    out_shape=jax.ShapeDtypeStruct((N, F), jnp.float32),
)


def _tc_layer1_body(d0, d1, a0, a1, xh, W1, b1, oA, oB):
    dinv = lax.rsqrt(d0[...] + d1[...] + 1.0)
    pre = (a0[...] + a1[...] + xh[...]) * dinv
    h = jnp.dot(pre, W1[...], preferred_element_type=jnp.float32) + b1[...]
    hh = jnp.maximum(h, 0.0) * dinv
    oA[...] = hh[:, :F]
    oB[...] = hh[:, F:]


_tc_layer1 = pl.pallas_call(
    _tc_layer1_body,
    grid=(N // BN,),
    in_specs=[
        pl.BlockSpec((BN, 1), lambda i: (i, 0)),
        pl.BlockSpec((BN, 1), lambda i: (i, 0)),
        pl.BlockSpec((BN, F), lambda i: (i, 0)),
        pl.BlockSpec((BN, F), lambda i: (i, 0)),
        pl.BlockSpec((BN, F), lambda i: (i, 0)),
        pl.BlockSpec((F, H), lambda i: (0, 0)),
        pl.BlockSpec((1, H), lambda i: (0, 0)),
    ],
    out_specs=(
        pl.BlockSpec((BN, F), lambda i: (i, 0)),
        pl.BlockSpec((BN, F), lambda i: (i, 0)),
    ),
    out_shape=(
        jax.ShapeDtypeStruct((N, F), jnp.float32),
        jax.ShapeDtypeStruct((N, F), jnp.float32),
    ),
)


def _tc_layer2_body(d0, d1, g0, g1, hA, hB, W2, b2, Wfc, bfc, o):
    dinv = lax.rsqrt(d0[...] + d1[...] + 1.0)
    preA = (g0[...] + hA[...]) * dinv
    preB = (g1[...] + hB[...]) * dinv
    pre = jnp.concatenate([preA, preB], axis=1)
    t = jnp.dot(pre, W2[...], preferred_element_type=jnp.float32) + b2[...]
    t = jnp.maximum(t, 0.0)
    logits = jnp.dot(t, Wfc[...], preferred_element_type=jnp.float32) + bfc[...]
    m = jnp.max(logits, axis=1, keepdims=True)
    e = jnp.exp(logits - m)
    o[...] = e / jnp.sum(e, axis=1, keepdims=True)


_tc_layer2 = pl.pallas_call(
    _tc_layer2_body,
    grid=(N // BN,),
    in_specs=[
        pl.BlockSpec((BN, 1), lambda i: (i, 0)),
        pl.BlockSpec((BN, 1), lambda i: (i, 0)),
        pl.BlockSpec((BN, F), lambda i: (i, 0)),
        pl.BlockSpec((BN, F), lambda i: (i, 0)),
        pl.BlockSpec((BN, F), lambda i: (i, 0)),
        pl.BlockSpec((BN, F), lambda i: (i, 0)),
        pl.BlockSpec((H, H), lambda i: (0, 0)),
        pl.BlockSpec((1, H), lambda i: (0, 0)),
        pl.BlockSpec((H, CLS), lambda i: (0, 0)),
        pl.BlockSpec((1, CLS), lambda i: (0, 0)),
    ],
    out_specs=pl.BlockSpec((BN, CLS), lambda i: (i, 0)),
    out_shape=jax.ShapeDtypeStruct((N, CLS), jnp.float32),
)


def kernel(x, edge_index, W1, b1, W2, b2, Wfc, bfc):
    src = edge_index[0].astype(jnp.int32)
    dst = edge_index[1].astype(jnp.int32)
    pad = EPAD - E
    src2d = jnp.concatenate([src, jnp.zeros((pad,), jnp.int32)]).reshape(ECH, CH)
    dst2d = jnp.concatenate([dst, jnp.full((pad,), N, jnp.int32)]).reshape(ECH, CH)
    srcs2 = jnp.concatenate([src2d, src2d + N], axis=0)
    ones8 = jnp.ones((CH, 16), jnp.float32)
    zer8 = jnp.zeros((NPAD, 16), jnp.float32)
    zerF = jnp.zeros((NPAD, F), jnp.float32)

    deg = _sc_degree(dst2d, ones8, zer8)
    d0 = deg[:N, :1]
    d1 = deg[NPAD:NPAD + N, :1]
    xh = _tc_prescale(d0, d1, x)
    agg = _sc_edge_sum_split(xh, src2d, dst2d, zerF)
    h1A, h1B = _tc_layer1(d0, d1, agg[:N], agg[NPAD:NPAD + N], xh, W1,
                          b1.reshape(1, H))
    tabs = jnp.concatenate([h1A, h1B], axis=0)
    g = _sc_edge_sum_sharded(tabs, srcs2, dst2d, zerF)
    return _tc_layer2(d0, d1, g[:N], g[NPAD:NPAD + N], h1A, h1B, W2,
                      b2.reshape(1, H), Wfc, bfc.reshape(1, CLS))


# speedup vs baseline: 10.0160x; 1.1016x over previous
import functools

import jax
import jax.numpy as jnp
from jax import lax
from jax.experimental import pallas as pl
from jax.experimental.pallas import tpu as pltpu
from jax.experimental.pallas import tpu_sc as plsc

N = 10000
F = 128
H = 256
CLS = 40
E = 320000
CH = 128
NT = 16
NC = 2
NPAD = 10112
RPT = NPAD // NT
ECH = 2560
EPAD = ECH * CH
NB = 16
NBF = 40

_MESH = plsc.VectorSubcoreMesh(
    core_axis_name="c", subcore_axis_name="s", num_cores=NC, num_subcores=NT)


def _edge_pass(tab, src_hbm, dst_hbm, acc, sidx, didx, rows, gsem,
               base, nblk):
    def block(bb, carry):
        off = base + bb * NBF
        pltpu.sync_copy(src_hbm.at[pl.ds(off, NBF)], sidx)
        pltpu.sync_copy(dst_hbm.at[pl.ds(off, NBF)], didx)
        pltpu.async_copy(tab.at[sidx.at[0]], rows.at[0], gsem)

        def ibody(j, carry2):
            b = lax.rem(j, 2)
            pltpu.make_async_copy(tab.at[sidx.at[j]], rows.at[b], gsem).wait()

            @pl.when(j + 1 < NBF)
            def _():
                pltpu.async_copy(tab.at[sidx.at[j + 1]], rows.at[1 - b], gsem)

            pltpu.sync_copy(rows.at[b], acc.at[didx.at[j]], add=True)
            return carry2

        lax.fori_loop(0, NBF, ibody, None)
        return carry

    lax.fori_loop(0, nblk, block, None)


@functools.partial(
    pl.kernel,
    out_type=jax.ShapeDtypeStruct((NC * NPAD, 16), jnp.float32),
    mesh=_MESH,
    scratch_types=[
        pltpu.VMEM((NB, CH), jnp.int32),
        pltpu.VMEM((CH, 16), jnp.float32),
        pltpu.VMEM_SHARED((NPAD, 16), jnp.float32),
    ],
    compiler_params=pltpu.CompilerParams(use_tc_tiling_on_sc=False),
)
def _sc_degree(dst_hbm, ones_hbm, zeros_hbm, out, didx, ones_v, acc):
    c = lax.axis_index("c")
    s = lax.axis_index("s")
    wid = c * NT + s
    r0 = s * RPT
    nch = ECH // (NC * NT)
    pltpu.sync_copy(zeros_hbm.at[pl.ds(r0, RPT)], acc.at[pl.ds(r0, RPT)])
    pltpu.sync_copy(ones_hbm, ones_v)
    plsc.subcore_barrier()

    def outer(b, carry):
        pltpu.sync_copy(dst_hbm.at[pl.ds(wid * nch + b * NB, NB)], didx)

        def body(j, carry2):
            pltpu.sync_copy(ones_v, acc.at[didx.at[j]], add=True)
            return carry2

        lax.fori_loop(0, NB, body, None)
        return carry

    lax.fori_loop(0, nch // NB, outer, None)
    plsc.subcore_barrier()
    pltpu.sync_copy(acc.at[pl.ds(r0, RPT)], out.at[pl.ds(c * NPAD + r0, RPT)])


@functools.partial(
    pl.kernel,
    out_type=jax.ShapeDtypeStruct((NC * NPAD, F), jnp.float32),
    mesh=_MESH,
    scratch_types=[
        pltpu.VMEM((NBF, CH), jnp.int32),
        pltpu.VMEM((NBF, CH), jnp.int32),
        pltpu.VMEM((2, CH, F), jnp.float32),
        pltpu.VMEM_SHARED((NPAD, F), jnp.float32),
        pltpu.SemaphoreType.DMA,
    ],
)
def _sc_edge_sum_split(tab_hbm, src_hbm, dst_hbm, zeros_hbm, out,
                       sidx, didx, rows, acc, gsem):
    c = lax.axis_index("c")
    s = lax.axis_index("s")
    wid = c * NT + s
    r0 = s * RPT
    nch = ECH // (NC * NT)
    pltpu.sync_copy(zeros_hbm.at[pl.ds(r0, RPT)], acc.at[pl.ds(r0, RPT)])
    plsc.subcore_barrier()
    _edge_pass(tab_hbm, src_hbm, dst_hbm, acc, sidx, didx, rows, gsem,
               wid * nch, nch // NBF)
    plsc.subcore_barrier()
    pltpu.sync_copy(acc.at[pl.ds(r0, RPT)], out.at[pl.ds(c * NPAD + r0, RPT)])


@functools.partial(
    pl.kernel,
    out_type=jax.ShapeDtypeStruct((NC * NPAD, F), jnp.float32),
    mesh=_MESH,
    scratch_types=[
        pltpu.VMEM((NBF, CH), jnp.int32),
        pltpu.VMEM((NBF, CH), jnp.int32),
        pltpu.VMEM((2, CH, F), jnp.float32),
        pltpu.VMEM_SHARED((NPAD, F), jnp.float32),
        pltpu.SemaphoreType.DMA,
    ],
)
def _sc_edge_sum_sharded(tabs_hbm, srcs_hbm, dsts_hbm, zeros_hbm, out,
                         sidx, didx, rows, acc, gsem):
    c = lax.axis_index("c")
    s = lax.axis_index("s")
    r0 = s * RPT
    nch = ECH // NT
    pltpu.sync_copy(zeros_hbm.at[pl.ds(r0, RPT)], acc.at[pl.ds(r0, RPT)])
    plsc.subcore_barrier()
    _edge_pass(tabs_hbm, srcs_hbm, dsts_hbm, acc, sidx, didx, rows, gsem,
               c * ECH + s * nch, nch // NBF)
    plsc.subcore_barrier()
    pltpu.sync_copy(acc.at[pl.ds(r0, RPT)], out.at[pl.ds(c * NPAD + r0, RPT)])


BN = 2000


def _tc_prescale_body(d0, d1, x, o):
    dinv = lax.rsqrt(d0[...] + d1[...] + 1.0)
    o[...] = x[...] * dinv


_tc_prescale = pl.pallas_call(
    _tc_prescale_body,
    grid=(N // BN,),
    in_specs=[
        pl.BlockSpec((BN, 1), lambda i: (i, 0)),
        pl.BlockSpec((BN, 1), lambda i: (i, 0)),
        pl.BlockSpec((BN, F), lambda i: (i, 0)),
    ],
    out_specs=pl.BlockSpec((BN, F), lambda i: (i, 0)),
    out_shape=jax.ShapeDtypeStruct((N, F), jnp.float32),
)


def _tc_layer1_body(d0, d1, a0, a1, xh, W1, b1, oA, oB):
    dinv = lax.rsqrt(d0[...] + d1[...] + 1.0)
    pre = (a0[...] + a1[...] + xh[...]) * dinv
    h = jnp.dot(pre, W1[...], preferred_element_type=jnp.float32) + b1[...]
    hh = jnp.maximum(h, 0.0) * dinv
    oA[...] = hh[:, :F]
    oB[...] = hh[:, F:]


_tc_layer1 = pl.pallas_call(
    _tc_layer1_body,
    grid=(N // BN,),
    in_specs=[
        pl.BlockSpec((BN, 1), lambda i: (i, 0)),
        pl.BlockSpec((BN, 1), lambda i: (i, 0)),
        pl.BlockSpec((BN, F), lambda i: (i, 0)),
        pl.BlockSpec((BN, F), lambda i: (i, 0)),
        pl.BlockSpec((BN, F), lambda i: (i, 0)),
        pl.BlockSpec((F, H), lambda i: (0, 0)),
        pl.BlockSpec((1, H), lambda i: (0, 0)),
    ],
    out_specs=(
        pl.BlockSpec((BN, F), lambda i: (i, 0)),
        pl.BlockSpec((BN, F), lambda i: (i, 0)),
    ),
    out_shape=(
        jax.ShapeDtypeStruct((N, F), jnp.float32),
        jax.ShapeDtypeStruct((N, F), jnp.float32),
    ),
)


def _tc_layer2_body(d0, d1, g0, g1, hA, hB, W2, b2, Wfc, bfc, o):
    dinv = lax.rsqrt(d0[...] + d1[...] + 1.0)
    preA = (g0[...] + hA[...]) * dinv
    preB = (g1[...] + hB[...]) * dinv
    pre = jnp.concatenate([preA, preB], axis=1)
    t = jnp.dot(pre, W2[...], preferred_element_type=jnp.float32) + b2[...]
    t = jnp.maximum(t, 0.0)
    logits = jnp.dot(t, Wfc[...], preferred_element_type=jnp.float32) + bfc[...]
    m = jnp.max(logits, axis=1, keepdims=True)
    e = jnp.exp(logits - m)
    o[...] = e / jnp.sum(e, axis=1, keepdims=True)


_tc_layer2 = pl.pallas_call(
    _tc_layer2_body,
    grid=(N // BN,),
    in_specs=[
        pl.BlockSpec((BN, 1), lambda i: (i, 0)),
        pl.BlockSpec((BN, 1), lambda i: (i, 0)),
        pl.BlockSpec((BN, F), lambda i: (i, 0)),
        pl.BlockSpec((BN, F), lambda i: (i, 0)),
        pl.BlockSpec((BN, F), lambda i: (i, 0)),
        pl.BlockSpec((BN, F), lambda i: (i, 0)),
        pl.BlockSpec((H, H), lambda i: (0, 0)),
        pl.BlockSpec((1, H), lambda i: (0, 0)),
        pl.BlockSpec((H, CLS), lambda i: (0, 0)),
        pl.BlockSpec((1, CLS), lambda i: (0, 0)),
    ],
    out_specs=pl.BlockSpec((BN, CLS), lambda i: (i, 0)),
    out_shape=jax.ShapeDtypeStruct((N, CLS), jnp.float32),
)


def kernel(x, edge_index, W1, b1, W2, b2, Wfc, bfc):
    src = edge_index[0].astype(jnp.int32)
    dst = edge_index[1].astype(jnp.int32)
    pad = EPAD - E
    src2d = jnp.concatenate([src, jnp.zeros((pad,), jnp.int32)]).reshape(ECH, CH)
    dst2d = jnp.concatenate([dst, jnp.full((pad,), N, jnp.int32)]).reshape(ECH, CH)
    srcs2 = jnp.concatenate([src2d, src2d + N], axis=0)
    dsts2 = jnp.concatenate([dst2d, dst2d], axis=0)
    ones8 = jnp.ones((CH, 16), jnp.float32)
    zer8 = jnp.zeros((NPAD, 16), jnp.float32)
    zerF = jnp.zeros((NPAD, F), jnp.float32)

    deg = _sc_degree(dst2d, ones8, zer8)
    d0 = deg[:N, :1]
    d1 = deg[NPAD:NPAD + N, :1]
    xh = _tc_prescale(d0, d1, x)
    agg = _sc_edge_sum_split(xh, src2d, dst2d, zerF)
    h1A, h1B = _tc_layer1(d0, d1, agg[:N], agg[NPAD:NPAD + N], xh, W1,
                          b1.reshape(1, H))
    tabs = jnp.concatenate([h1A, h1B], axis=0)
    g = _sc_edge_sum_sharded(tabs, srcs2, dsts2, zerF)
    return _tc_layer2(d0, d1, g[:N], g[NPAD:NPAD + N], h1A, h1B, W2,
                      b2.reshape(1, H), Wfc, bfc.reshape(1, CLS))


# speedup vs baseline: 23.4922x; 2.3455x over previous
import functools

import jax
import jax.numpy as jnp
from jax import lax
from jax.experimental import pallas as pl
from jax.experimental.pallas import tpu as pltpu
from jax.experimental.pallas import tpu_sc as plsc

N = 10000
F = 128
H = 256
CLS = 40
E = 320000
CH = 128
NT = 16
NC = 2
NPAD = 10112
RPT = NPAD // NT
ECH = 2560
EPAD = ECH * CH
NB = 16
NBF = 40

_MESH = plsc.VectorSubcoreMesh(
    core_axis_name="c", subcore_axis_name="s", num_cores=NC, num_subcores=NT)


def _edge_pass(tab, src_hbm, dst_hbm, acc, sidx, didx, rows, gsem,
               base, nblk):
    def block(bb, carry):
        off = base + bb * NBF
        pltpu.sync_copy(src_hbm.at[pl.ds(off, NBF)], sidx)
        pltpu.sync_copy(dst_hbm.at[pl.ds(off, NBF)], didx)
        pltpu.async_copy(tab.at[sidx.at[0]], rows.at[0], gsem)

        def ibody(j, carry2):
            b = lax.rem(j, 2)
            pltpu.make_async_copy(tab.at[sidx.at[j]], rows.at[b], gsem).wait()

            @pl.when(j + 1 < NBF)
            def _():
                pltpu.async_copy(tab.at[sidx.at[j + 1]], rows.at[1 - b], gsem)

            pltpu.sync_copy(rows.at[b], acc.at[didx.at[j]], add=True)
            return carry2

        lax.fori_loop(0, NBF, ibody, None)
        return carry

    lax.fori_loop(0, nblk, block, None)


@functools.partial(
    pl.kernel,
    out_type=jax.ShapeDtypeStruct((NC * NPAD, 16), jnp.float32),
    mesh=_MESH,
    scratch_types=[
        pltpu.VMEM((NB, CH), jnp.int32),
        pltpu.VMEM((CH, 16), jnp.float32),
        pltpu.VMEM_SHARED((NPAD, 16), jnp.float32),
    ],
    compiler_params=pltpu.CompilerParams(use_tc_tiling_on_sc=False),
)
def _sc_degree(dst_hbm, ones_hbm, zeros_hbm, out, didx, ones_v, acc):
    c = lax.axis_index("c")
    s = lax.axis_index("s")
    wid = c * NT + s
    r0 = s * RPT
    nch = ECH // (NC * NT)
    pltpu.sync_copy(zeros_hbm.at[pl.ds(r0, RPT)], acc.at[pl.ds(r0, RPT)])
    pltpu.sync_copy(ones_hbm, ones_v)
    plsc.subcore_barrier()

    def outer(b, carry):
        pltpu.sync_copy(dst_hbm.at[pl.ds(wid * nch + b * NB, NB)], didx)

        def body(j, carry2):
            pltpu.sync_copy(ones_v, acc.at[didx.at[j]], add=True)
            return carry2

        lax.fori_loop(0, NB, body, None)
        return carry

    lax.fori_loop(0, nch // NB, outer, None)
    plsc.subcore_barrier()
    pltpu.sync_copy(acc.at[pl.ds(r0, RPT)], out.at[pl.ds(c * NPAD + r0, RPT)])


@functools.partial(
    pl.kernel,
    out_type=jax.ShapeDtypeStruct((NC * NPAD, F), jnp.float32),
    mesh=_MESH,
    scratch_types=[
        pltpu.VMEM((NBF, CH), jnp.int32),
        pltpu.VMEM((NBF, CH), jnp.int32),
        pltpu.VMEM((2, CH, F), jnp.float32),
        pltpu.VMEM_SHARED((NPAD, F), jnp.float32),
        pltpu.SemaphoreType.DMA,
    ],
)
def _sc_edge_sum_split(tab_hbm, src_hbm, dst_hbm, zeros_hbm, out,
                       sidx, didx, rows, acc, gsem):
    c = lax.axis_index("c")
    s = lax.axis_index("s")
    wid = c * NT + s
    r0 = s * RPT
    nch = ECH // (NC * NT)
    pltpu.sync_copy(zeros_hbm.at[pl.ds(r0, RPT)], acc.at[pl.ds(r0, RPT)])
    plsc.subcore_barrier()
    _edge_pass(tab_hbm, src_hbm, dst_hbm, acc, sidx, didx, rows, gsem,
               wid * nch, nch // NBF)
    plsc.subcore_barrier()
    pltpu.sync_copy(acc.at[pl.ds(r0, RPT)], out.at[pl.ds(c * NPAD + r0, RPT)])


@functools.partial(
    pl.kernel,
    out_type=jax.ShapeDtypeStruct((NC * NPAD, F), jnp.float32),
    mesh=_MESH,
    scratch_types=[
        pltpu.VMEM((NBF, CH), jnp.int32),
        pltpu.VMEM((NBF, CH), jnp.int32),
        pltpu.VMEM((2, CH, F), jnp.float32),
        pltpu.VMEM_SHARED((NPAD, F), jnp.float32),
        pltpu.SemaphoreType.DMA,
    ],
)
def _sc_edge_sum_sharded(tabs_hbm, srcs_hbm, dsts_hbm, zeros_hbm, out,
                         sidx, didx, rows, acc, gsem):
    c = lax.axis_index("c")
    s = lax.axis_index("s")
    r0 = s * RPT
    nch = ECH // NT
    pltpu.sync_copy(zeros_hbm.at[pl.ds(r0, RPT)], acc.at[pl.ds(r0, RPT)])
    plsc.subcore_barrier()
    _edge_pass(tabs_hbm, srcs_hbm, dsts_hbm, acc, sidx, didx, rows, gsem,
               c * ECH + s * nch, nch // NBF)
    plsc.subcore_barrier()
    pltpu.sync_copy(acc.at[pl.ds(r0, RPT)], out.at[pl.ds(c * NPAD + r0, RPT)])


BN = 2000


def _tc_prescale_body(d0, d1, x, o):
    dinv = lax.rsqrt(d0[...] + d1[...] + 1.0)
    o[...] = x[...] * dinv


_tc_prescale = pl.pallas_call(
    _tc_prescale_body,
    grid=(N // BN,),
    in_specs=[
        pl.BlockSpec((BN, 1), lambda i: (i, 0)),
        pl.BlockSpec((BN, 1), lambda i: (i, 0)),
        pl.BlockSpec((BN, F), lambda i: (i, 0)),
    ],
    out_specs=pl.BlockSpec((BN, F), lambda i: (i, 0)),
    out_shape=jax.ShapeDtypeStruct((N, F), jnp.float32),
)


def _tc_layer1_body(d0, d1, a0, a1, xh, W1, b1, oA, oB):
    dinv = lax.rsqrt(d0[...] + d1[...] + 1.0)
    pre = (a0[...] + a1[...] + xh[...]) * dinv
    h = jnp.dot(pre, W1[...], preferred_element_type=jnp.float32) + b1[...]
    hh = jnp.maximum(h, 0.0) * dinv
    oA[...] = hh[:, :F]
    oB[...] = hh[:, F:]


_tc_layer1 = pl.pallas_call(
    _tc_layer1_body,
    grid=(N // BN,),
    in_specs=[
        pl.BlockSpec((BN, 1), lambda i: (i, 0)),
        pl.BlockSpec((BN, 1), lambda i: (i, 0)),
        pl.BlockSpec((BN, F), lambda i: (i, 0)),
        pl.BlockSpec((BN, F), lambda i: (i, 0)),
        pl.BlockSpec((BN, F), lambda i: (i, 0)),
        pl.BlockSpec((F, H), lambda i: (0, 0)),
        pl.BlockSpec((1, H), lambda i: (0, 0)),
    ],
    out_specs=(
        pl.BlockSpec((BN, F), lambda i: (i, 0)),
        pl.BlockSpec((BN, F), lambda i: (i, 0)),
    ),
    out_shape=(
        jax.ShapeDtypeStruct((N, F), jnp.float32),
        jax.ShapeDtypeStruct((N, F), jnp.float32),
    ),
)


def _tc_layer2_body(d0, d1, g0, g1, hA, hB, W2, b2, Wfc, bfc, o):
    dinv = lax.rsqrt(d0[...] + d1[...] + 1.0)
    preA = (g0[...] + hA[...]) * dinv
    preB = (g1[...] + hB[...]) * dinv
    pre = jnp.concatenate([preA, preB], axis=1)
    t = jnp.dot(pre, W2[...], preferred_element_type=jnp.float32) + b2[...]
    t = jnp.maximum(t, 0.0)
    logits = jnp.dot(t, Wfc[...], preferred_element_type=jnp.float32) + bfc[...]
    m = jnp.max(logits, axis=1, keepdims=True)
    e = jnp.exp(logits - m)
    o[...] = e / jnp.sum(e, axis=1, keepdims=True)


_tc_layer2 = pl.pallas_call(
    _tc_layer2_body,
    grid=(N // BN,),
    in_specs=[
        pl.BlockSpec((BN, 1), lambda i: (i, 0)),
        pl.BlockSpec((BN, 1), lambda i: (i, 0)),
        pl.BlockSpec((BN, F), lambda i: (i, 0)),
        pl.BlockSpec((BN, F), lambda i: (i, 0)),
        pl.BlockSpec((BN, F), lambda i: (i, 0)),
        pl.BlockSpec((BN, F), lambda i: (i, 0)),
        pl.BlockSpec((H, H), lambda i: (0, 0)),
        pl.BlockSpec((1, H), lambda i: (0, 0)),
        pl.BlockSpec((H, CLS), lambda i: (0, 0)),
        pl.BlockSpec((1, CLS), lambda i: (0, 0)),
    ],
    out_specs=pl.BlockSpec((BN, CLS), lambda i: (i, 0)),
    out_shape=jax.ShapeDtypeStruct((N, CLS), jnp.float32),
)


def kernel(x, edge_index, W1, b1, W2, b2, Wfc, bfc):
    src = edge_index[0].astype(jnp.int32)
    dst = edge_index[1].astype(jnp.int32)
    pad = EPAD - E
    pad_src = jnp.arange(pad, dtype=jnp.int32) % N
    pad_dst = N + jnp.arange(pad, dtype=jnp.int32) % (NPAD - N)
    src2d = jnp.concatenate([src, pad_src]).reshape(ECH, CH)
    dst2d = jnp.concatenate([dst, pad_dst]).reshape(ECH, CH)
    srcs2 = jnp.concatenate([src2d, src2d + N], axis=0)
    dsts2 = jnp.concatenate([dst2d, dst2d], axis=0)
    ones8 = jnp.ones((CH, 16), jnp.float32)
    zer8 = jnp.zeros((NPAD, 16), jnp.float32)
    zerF = jnp.zeros((NPAD, F), jnp.float32)

    deg = _sc_degree(dst2d, ones8, zer8)
    d0 = deg[:N, :1]
    d1 = deg[NPAD:NPAD + N, :1]
    xh = _tc_prescale(d0, d1, x)
    agg = _sc_edge_sum_split(xh, src2d, dst2d, zerF)
    h1A, h1B = _tc_layer1(d0, d1, agg[:N], agg[NPAD:NPAD + N], xh, W1,
                          b1.reshape(1, H))
    tabs = jnp.concatenate([h1A, h1B], axis=0)
    g = _sc_edge_sum_sharded(tabs, srcs2, dsts2, zerF)
    return _tc_layer2(d0, d1, g[:N], g[NPAD:NPAD + N], h1A, h1B, W2,
                      b2.reshape(1, H), Wfc, bfc.reshape(1, CLS))
